# speedup vs baseline: 5.3788x; 5.3788x over previous
import functools

import jax
import jax.numpy as jnp
from jax import lax
from jax.experimental import pallas as pl
from jax.experimental.pallas import tpu as pltpu
from jax.experimental.pallas import tpu_sc as plsc

_N, _E, _D = 10000, 320000, 128
_NC, _NS = 2, 16
_NW = _NC * _NS
_CH = 128
_CPW = 79
_EPW = _CH * _CPW
_E_PAD = _NW * _EPW
_N_PAD = 10240
_RPS = _N_PAD // _NS


def _sc_gather_segsum(x, src, dst):
    mesh = plsc.VectorSubcoreMesh(
        core_axis_name="c", subcore_axis_name="s",
        num_cores=_NC, num_subcores=_NS)

    @functools.partial(
        pl.kernel,
        out_type=(jax.ShapeDtypeStruct((_NC, _N_PAD, _D), jnp.float32),
                  jax.ShapeDtypeStruct((_NC, _N_PAD), jnp.float32)),
        mesh=mesh,
        scratch_types=[
            pltpu.VMEM_SHARED((_N_PAD, _D), jnp.float32),
            pltpu.VMEM_SHARED((_N_PAD,), jnp.float32),
            pltpu.VMEM((_CH,), jnp.int32),
            pltpu.VMEM((_CH,), jnp.int32),
            pltpu.VMEM((_CH, _D), jnp.float32),
            pltpu.VMEM((_CH,), jnp.float32),
            pltpu.VMEM((16, _D), jnp.float32),
            pltpu.VMEM((_RPS,), jnp.float32),
            pltpu.SemaphoreType.DMA,
        ])
    def k(x_hbm, src_hbm, dst_hbm, acc_out, deg_out,
          acc_sh, deg_sh, src_v, dst_v, rows_v, ones_v, zrow_v, zcol_v, sem):
        cid = lax.axis_index("c")
        sid = lax.axis_index("s")
        wid = sid * _NC + cid

        zero16 = jnp.zeros((16,), jnp.float32)
        one16 = jnp.ones((16,), jnp.float32)

        def fill_zrow(i, carry):
            r = i // (_D // 16)
            c = (i % (_D // 16)) * 16
            zrow_v[r, pl.ds(c, 16)] = zero16
            return carry
        lax.fori_loop(0, 16 * (_D // 16), fill_zrow, 0)

        def fill_zcol(i, carry):
            zcol_v[pl.ds(i * 16, 16)] = zero16
            return carry
        lax.fori_loop(0, _RPS // 16, fill_zcol, 0)

        def fill_ones(i, carry):
            ones_v[pl.ds(i * 16, 16)] = one16
            return carry
        lax.fori_loop(0, _CH // 16, fill_ones, 0)

        rbase = sid * _RPS

        def zero_acc(i, carry):
            pltpu.sync_copy(zrow_v, acc_sh.at[pl.ds(rbase + i * 16, 16)])
            return carry
        lax.fori_loop(0, _RPS // 16, zero_acc, 0)
        pltpu.sync_copy(zcol_v, deg_sh.at[pl.ds(rbase, _RPS)])

        plsc.subcore_barrier()

        ebase = wid * _EPW

        def edge_body(c, carry):
            base = ebase + c * _CH
            pltpu.sync_copy(src_hbm.at[pl.ds(base, _CH)], src_v)
            pltpu.sync_copy(dst_hbm.at[pl.ds(base, _CH)], dst_v)
            pltpu.async_copy(x_hbm.at[src_v], rows_v, sem).wait()
            pltpu.sync_copy(rows_v, acc_sh.at[dst_v], add=True)
            pltpu.sync_copy(ones_v, deg_sh.at[dst_v], add=True)
            return carry
        lax.fori_loop(0, _CPW, edge_body, 0)

        plsc.subcore_barrier()

        pltpu.sync_copy(acc_sh.at[pl.ds(rbase, _RPS)],
                        acc_out.at[cid, pl.ds(rbase, _RPS)])
        pltpu.sync_copy(deg_sh.at[pl.ds(rbase, _RPS)],
                        deg_out.at[cid, pl.ds(rbase, _RPS)])

    return k(x, src, dst)


def _tc_combine(x_pad, parts_flat, degs_flat, W_self, W_neigh, b2):
    bn = 512
    g = _N_PAD // bn

    def body(x_ref, p0_ref, p1_ref, d0_ref, d1_ref, ws_ref, wn_ref, b_ref,
             o_ref):
        deg = jnp.maximum(d0_ref[...] + d1_ref[...], 1.0)
        h = (p0_ref[...] + p1_ref[...]) / deg[:, None]
        out = (jnp.dot(x_ref[...], ws_ref[...],
                       preferred_element_type=jnp.float32)
               + jnp.dot(h, wn_ref[...], preferred_element_type=jnp.float32)
               + b_ref[...])
        o_ref[...] = jnp.maximum(out, 0.0)

    return pl.pallas_call(
        body,
        grid=(g,),
        in_specs=[
            pl.BlockSpec((bn, _D), lambda i: (i, 0)),
            pl.BlockSpec((bn, _D), lambda i: (i, 0)),
            pl.BlockSpec((bn, _D), lambda i: (i + g, 0)),
            pl.BlockSpec((bn,), lambda i: (i,)),
            pl.BlockSpec((bn,), lambda i: (i + g,)),
            pl.BlockSpec((_D, _D), lambda i: (0, 0)),
            pl.BlockSpec((_D, _D), lambda i: (0, 0)),
            pl.BlockSpec((1, _D), lambda i: (0, 0)),
        ],
        out_specs=pl.BlockSpec((bn, _D), lambda i: (i, 0)),
        out_shape=jax.ShapeDtypeStruct((_N_PAD, _D), jnp.float32),
    )(x_pad, parts_flat, parts_flat, degs_flat, degs_flat,
      W_self, W_neigh, b2)


def kernel(x, edge_index, W_self, W_neigh, b):
    src = edge_index[0]
    dst = edge_index[1]
    npad = _E_PAD - _E
    src_p = jnp.concatenate([src, jnp.zeros((npad,), jnp.int32)])
    dst_p = jnp.concatenate([dst, jnp.full((npad,), _N, jnp.int32)])
    acc, deg = _sc_gather_segsum(x, src_p, dst_p)
    x_pad = jnp.concatenate([x, jnp.zeros((_N_PAD - _N, _D), x.dtype)])
    out = _tc_combine(x_pad, acc.reshape(-1, _D), deg.reshape(-1),
                      W_self, W_neigh, b.reshape(1, _D))
    return out[:_N]
